# baseline (speedup 1.0000x reference)
import jax
import jax.numpy as jnp
from jax.experimental import pallas as pl
from jax.experimental.pallas import tpu as pltpu

_D = 4
_CB = 4


def _in_copy(prob_ref, in_bufs, in_sems, chunk, slot):
    return pltpu.make_async_copy(
        prob_ref.at[pl.ds(chunk * _CB, _CB), :],
        in_bufs.at[slot],
        in_sems.at[slot],
    )


def _out_copy(out_ref, out_bufs, out_sems, chunk, slot):
    return pltpu.make_async_copy(
        out_bufs.at[slot],
        out_ref.at[pl.ds(chunk * _CB, _CB), :],
        out_sems.at[slot],
    )


def _kern(prob_ref, out_ref, in_bufs, out_bufs, in_sems, out_sems):
    i = pl.program_id(0)
    n = pl.num_programs(0)
    slot = jax.lax.rem(i, _D)

    @pl.when(i == 0)
    def _pro():
        for j in range(_D):
            _in_copy(prob_ref, in_bufs, in_sems, j, j).start()

    _in_copy(prob_ref, in_bufs, in_sems, i, slot).wait()

    @pl.when(i >= _D)
    def _drain():
        _out_copy(out_ref, out_bufs, out_sems, i - _D, slot).wait()

    out_bufs[slot] = in_bufs[slot]

    _out_copy(out_ref, out_bufs, out_sems, i, slot).start()

    @pl.when(i + _D < n)
    def _re():
        _in_copy(prob_ref, in_bufs, in_sems, i + _D, slot).start()

    @pl.when(i == n - 1)
    def _epi():
        for j in range(_D):
            c = n - _D + j
            _out_copy(out_ref, out_bufs, out_sems, c,
                      jax.lax.rem(jnp.int32(c), _D)).wait()


def kernel(probabilites, mask, step):
    del mask, step
    B, V = probabilites.shape
    W = 400000
    R = B * V // W
    pv = probabilites.reshape(R, W)
    n_chunks = R // _CB
    out = pl.pallas_call(
        _kern,
        grid=(n_chunks,),
        in_specs=[pl.BlockSpec(memory_space=pl.ANY)],
        out_specs=pl.BlockSpec(memory_space=pl.ANY),
        out_shape=jax.ShapeDtypeStruct((R, W), probabilites.dtype),
        scratch_shapes=[
            pltpu.VMEM((_D, _CB, W), jnp.float32),
            pltpu.VMEM((_D, _CB, W), jnp.float32),
            pltpu.SemaphoreType.DMA((_D,)),
            pltpu.SemaphoreType.DMA((_D,)),
        ],
    )(pv)
    return out.reshape(B, V)


# speedup vs baseline: 6.2751x; 6.2751x over previous
import jax
import jax.numpy as jnp
from jax.experimental import pallas as pl
from jax.experimental.pallas import tpu as pltpu

_BV = 1024


def _mask_mul_kernel(step_ref, prob_ref, mask_ref, out_ref):
    col = step_ref[0] - 1
    mblk = mask_ref[...]
    sel = jax.lax.broadcasted_iota(jnp.int32, mblk.shape, 1) == col
    m = jnp.sum(jnp.where(sel, mblk, 0.0), axis=1, keepdims=True)
    out_ref[...] = prob_ref[...] * m


def kernel(probabilites, mask, step):
    B, V = probabilites.shape
    S = mask.shape[0]
    pt = probabilites.T
    mt = mask.T
    step_arr = jnp.atleast_1d(jnp.asarray(step, jnp.int32))
    grid = ((V + _BV - 1) // _BV,)
    grid_spec = pltpu.PrefetchScalarGridSpec(
        num_scalar_prefetch=1,
        grid=grid,
        in_specs=[
            pl.BlockSpec((_BV, B), lambda i, s: (i, 0)),
            pl.BlockSpec((_BV, S), lambda i, s: (i, 0)),
        ],
        out_specs=pl.BlockSpec((_BV, B), lambda i, s: (i, 0)),
    )
    out_t = pl.pallas_call(
        _mask_mul_kernel,
        grid_spec=grid_spec,
        out_shape=jax.ShapeDtypeStruct((V, B), probabilites.dtype),
    )(step_arr, pt, mt)
    return out_t.T


# speedup vs baseline: 6.3111x; 1.0057x over previous
import jax
import jax.numpy as jnp
from jax.experimental import pallas as pl
from jax.experimental.pallas import tpu as pltpu

_BV = 2048


def _mask_mul_kernel(step_ref, prob_ref, mask_ref, out_ref):
    col = step_ref[0] - 1
    mblk = mask_ref[...]
    sel = jax.lax.broadcasted_iota(jnp.int32, mblk.shape, 1) == col
    m = jnp.sum(jnp.where(sel, mblk, 0.0), axis=1, keepdims=True)
    out_ref[...] = prob_ref[...] * m


def kernel(probabilites, mask, step):
    B, V = probabilites.shape
    S = mask.shape[0]
    pt = probabilites.T
    mt = mask.T
    step_arr = jnp.atleast_1d(jnp.asarray(step, jnp.int32))
    grid = ((V + _BV - 1) // _BV,)
    grid_spec = pltpu.PrefetchScalarGridSpec(
        num_scalar_prefetch=1,
        grid=grid,
        in_specs=[
            pl.BlockSpec((_BV, B), lambda i, s: (i, 0)),
            pl.BlockSpec((_BV, S), lambda i, s: (i, 0)),
        ],
        out_specs=pl.BlockSpec((_BV, B), lambda i, s: (i, 0)),
    )
    out_t = pl.pallas_call(
        _mask_mul_kernel,
        grid_spec=grid_spec,
        out_shape=jax.ShapeDtypeStruct((V, B), probabilites.dtype),
    )(step_arr, pt, mt)
    return out_t.T


# speedup vs baseline: 7.4404x; 1.1789x over previous
import jax
import jax.numpy as jnp
from jax.experimental import pallas as pl
from jax.experimental.pallas import tpu as pltpu

_BV = 2048


def _mask_mul_kernel(step_ref, prob_ref, mask_ref, out_ref):
    col = step_ref[0] - 1
    mt = jnp.transpose(mask_ref[...], (1, 0))
    sel = jax.lax.broadcasted_iota(jnp.int32, mt.shape, 1) == col
    m = jnp.sum(jnp.where(sel, mt, 0.0), axis=1, keepdims=True)
    out_ref[...] = prob_ref[...] * m


def kernel(probabilites, mask, step):
    B, V = probabilites.shape
    S = mask.shape[0]
    pt = probabilites.T
    step_arr = jnp.atleast_1d(jnp.asarray(step, jnp.int32))
    grid = ((V + _BV - 1) // _BV,)
    grid_spec = pltpu.PrefetchScalarGridSpec(
        num_scalar_prefetch=1,
        grid=grid,
        in_specs=[
            pl.BlockSpec((_BV, B), lambda i, s: (i, 0)),
            pl.BlockSpec((S, _BV), lambda i, s: (0, i)),
        ],
        out_specs=pl.BlockSpec((_BV, B), lambda i, s: (i, 0)),
    )
    out_t = pl.pallas_call(
        _mask_mul_kernel,
        grid_spec=grid_spec,
        out_shape=jax.ShapeDtypeStruct((V, B), probabilites.dtype),
    )(step_arr, pt, mask)
    return out_t.T


# speedup vs baseline: 7.4773x; 1.0050x over previous
import jax
import jax.numpy as jnp
from jax.experimental import pallas as pl
from jax.experimental.pallas import tpu as pltpu

_BV = 3072


def _mask_mul_kernel(step_ref, prob_ref, mask_ref, out_ref):
    col = step_ref[0] - 1
    mt = jnp.transpose(mask_ref[...], (1, 0))
    sel = jax.lax.broadcasted_iota(jnp.int32, mt.shape, 1) == col
    m = jnp.sum(jnp.where(sel, mt, 0.0), axis=1, keepdims=True)
    out_ref[...] = prob_ref[...] * m


def kernel(probabilites, mask, step):
    B, V = probabilites.shape
    S = mask.shape[0]
    pt = probabilites.T
    step_arr = jnp.atleast_1d(jnp.asarray(step, jnp.int32))
    grid = ((V + _BV - 1) // _BV,)
    grid_spec = pltpu.PrefetchScalarGridSpec(
        num_scalar_prefetch=1,
        grid=grid,
        in_specs=[
            pl.BlockSpec((_BV, B), lambda i, s: (i, 0)),
            pl.BlockSpec((S, _BV), lambda i, s: (0, i)),
        ],
        out_specs=pl.BlockSpec((_BV, B), lambda i, s: (i, 0)),
    )
    out_t = pl.pallas_call(
        _mask_mul_kernel,
        grid_spec=grid_spec,
        out_shape=jax.ShapeDtypeStruct((V, B), probabilites.dtype),
    )(step_arr, pt, mask)
    return out_t.T


# speedup vs baseline: 7.5342x; 1.0076x over previous
import jax
import jax.numpy as jnp
from jax.experimental import pallas as pl
from jax.experimental.pallas import tpu as pltpu

_BV = 3072


def _mask_mul_kernel(step_ref, prob_ref, mask_ref, out_ref):
    col = jax.lax.rem(step_ref[0] - 1, 8)
    mt = jnp.transpose(mask_ref[...], (1, 0))
    sel = jax.lax.broadcasted_iota(jnp.int32, mt.shape, 1) == col
    m = jnp.sum(jnp.where(sel, mt, 0.0), axis=1, keepdims=True)
    out_ref[...] = prob_ref[...] * m


def kernel(probabilites, mask, step):
    B, V = probabilites.shape
    S = mask.shape[0]
    pt = probabilites.T
    step_arr = jnp.atleast_1d(jnp.asarray(step, jnp.int32))
    grid = ((V + _BV - 1) // _BV,)
    grid_spec = pltpu.PrefetchScalarGridSpec(
        num_scalar_prefetch=1,
        grid=grid,
        in_specs=[
            pl.BlockSpec((_BV, B), lambda i, s: (i, 0)),
            pl.BlockSpec((8, _BV), lambda i, s: ((s[0] - 1) // 8, i)),
        ],
        out_specs=pl.BlockSpec((_BV, B), lambda i, s: (i, 0)),
    )
    out_t = pl.pallas_call(
        _mask_mul_kernel,
        grid_spec=grid_spec,
        out_shape=jax.ShapeDtypeStruct((V, B), probabilites.dtype),
    )(step_arr, pt, mask)
    return out_t.T
